# Initial kernel scaffold; baseline (speedup 1.0000x reference)
import jax
import jax.numpy as jnp
from jax.experimental import pallas as pl


def kernel(x, edge_index, edge_type, batch, stack_mask, W1, root1, b1, W2, root2, b2, stack_table):
    raise NotImplementedError("write your pallas kernel here")


# speedup vs baseline: 21.6047x; 21.6047x over previous
import functools
import jax
import jax.numpy as jnp
from jax import lax
from jax.experimental import pallas as pl
from jax.experimental.pallas import tpu as pltpu
from jax.experimental.pallas import tpu_sc as plsc

N = 10000
E = 320000
D = 128
R = 8
G = 128

NC, NS, L = 2, 16, 16
NW = NC * NS
EPW = E // NW
CH = 80
NCHK = EPW // CH
BLK = 25
NBLK = NCHK // BLK
WRB = 624
WRH = 48
REM = N - NS * WRB
KPT = (R * N) // NS

_f32 = jnp.float32


def _nan_clean(t):
    t = jnp.where(jnp.isnan(t), 0.0, t)
    t = jnp.where(t == jnp.inf, 1000000.0, t)
    t = jnp.where(t == -jnp.inf, -1000000.0, t)
    return t


_MESH = plsc.VectorSubcoreMesh(core_axis_name="c", subcore_axis_name="s")


@functools.partial(
    pl.kernel,
    out_type=jax.ShapeDtypeStruct((NC * R * N,), _f32),
    mesh=_MESH,
    scratch_types=[
        pltpu.VMEM((NBLK, BLK, CH), jnp.int32),
        pltpu.VMEM((CH,), _f32),
        pltpu.VMEM((KPT,), _f32),
        pltpu.VMEM_SHARED((R * N,), _f32),
    ],
)
def _k_deg(gdk_hbm, zeros_hbm, deg_out, idx_v, ones_v, zb_v, acc_sh):
    cid = lax.axis_index("c")
    sid = lax.axis_index("s")
    wid = sid * NC + cid

    pltpu.sync_copy(zeros_hbm, zb_v)
    pltpu.sync_copy(zb_v, acc_sh.at[pl.ds(sid * KPT, KPT)])

    for i in range(CH // L):
        ones_v[pl.ds(i * L, L)] = jnp.ones((L,), _f32)
    pltpu.sync_copy(gdk_hbm.at[wid], idx_v)
    plsc.subcore_barrier()

    def body(b, carry):
        def inner(j, c1):
            pltpu.sync_copy(ones_v, acc_sh.at[idx_v.at[b, j]], add=True)
            return c1
        return lax.fori_loop(0, BLK, inner, carry)

    lax.fori_loop(0, NBLK, body, 0)
    plsc.subcore_barrier()

    pltpu.sync_copy(acc_sh.at[pl.ds(sid * KPT, KPT)], zb_v)
    pltpu.sync_copy(zb_v, deg_out.at[pl.ds(cid * (R * N) + sid * KPT, KPT)])


@functools.partial(
    pl.kernel,
    out_type=jax.ShapeDtypeStruct((NW, NBLK, BLK, CH), _f32),
    mesh=_MESH,
    scratch_types=[
        pltpu.VMEM((NBLK, BLK, CH), jnp.int32),
        pltpu.VMEM((CH,), _f32),
        pltpu.VMEM((CH,), _f32),
        pltpu.VMEM((NBLK, BLK, CH), _f32),
        pltpu.SemaphoreType.DMA,
    ],
)
def _k_w(gdk_hbm, deg0_hbm, deg1_hbm, w_out, idx_v, d0_v, d1_v, w_v, sem):
    cid = lax.axis_index("c")
    sid = lax.axis_index("s")
    wid = sid * NC + cid
    pltpu.sync_copy(gdk_hbm.at[wid], idx_v)

    def body(b, carry):
        def inner(j, c1):
            pltpu.async_copy(deg0_hbm.at[idx_v.at[b, j]], d0_v, sem).wait()
            pltpu.async_copy(deg1_hbm.at[idx_v.at[b, j]], d1_v, sem).wait()
            for i in range(CH // L):
                sl = pl.ds(i * L, L)
                w_v[b, j, sl] = 1.0 / (d0_v[sl] + d1_v[sl])
            return c1
        return lax.fori_loop(0, BLK, inner, carry)

    lax.fori_loop(0, NBLK, body, 0)
    pltpu.sync_copy(w_v, w_out.at[wid])


@functools.partial(
    pl.kernel,
    out_type=jax.ShapeDtypeStruct((NC, N, D), _f32),
    mesh=_MESH,
    scratch_types=[
        pltpu.VMEM((BLK, CH), jnp.int32),
        pltpu.VMEM((BLK, CH), jnp.int32),
        pltpu.VMEM((BLK, CH), _f32),
        pltpu.VMEM((CH, D), _f32),
        pltpu.VMEM_SHARED((N, D), _f32),
        pltpu.SemaphoreType.DMA,
    ],
)
def _k_msg(table_hbm, gsrc_hbm, dst_hbm, w_hbm, zeros_hbm, part_out,
           gsrc_v, dst_v, w_v, rows_v, acc_sh, sem):
    cid = lax.axis_index("c")
    sid = lax.axis_index("s")
    wid = sid * NC + cid
    zb = rows_v.at[pl.ds(0, WRH)]

    pltpu.sync_copy(zeros_hbm, zb)
    for i in range(WRB // WRH):
        pltpu.sync_copy(zb, acc_sh.at[pl.ds(sid * WRB + i * WRH, WRH)])

    @pl.when(sid == 0)
    def _():
        pltpu.sync_copy(rows_v.at[pl.ds(0, REM)], acc_sh.at[pl.ds(NS * WRB, REM)])

    plsc.subcore_barrier()

    def blk(b, carry):
        pltpu.sync_copy(gsrc_hbm.at[wid, b], gsrc_v)
        pltpu.sync_copy(dst_hbm.at[wid, b], dst_v)
        pltpu.sync_copy(w_hbm.at[wid, b], w_v)

        def chunk(j, c1):
            pltpu.async_copy(table_hbm.at[gsrc_v.at[j]], rows_v, sem).wait()

            def scale(g, c2):
                wv = w_v[j, pl.ds(g * L, L)]
                for el in range(L):
                    e = g * L + el
                    w = wv[el]
                    for g2 in range(D // L):
                        sl = pl.ds(g2 * L, L)
                        rows_v[e, sl] = rows_v[e, sl] * w
                return c2

            lax.fori_loop(0, CH // L, scale, 0)
            pltpu.sync_copy(rows_v, acc_sh.at[dst_v.at[j]], add=True)
            return c1

        return lax.fori_loop(0, BLK, chunk, carry)

    lax.fori_loop(0, NBLK, blk, 0)
    plsc.subcore_barrier()

    for i in range(WRB // WRH):
        sl = pl.ds(sid * WRB + i * WRH, WRH)
        pltpu.sync_copy(acc_sh.at[sl], zb)
        pltpu.sync_copy(zb, part_out.at[cid, sl])

    @pl.when(sid == 0)
    def _():
        sl = pl.ds(NS * WRB, REM)
        pltpu.sync_copy(acc_sh.at[sl], rows_v.at[pl.ds(0, REM)])
        pltpu.sync_copy(rows_v.at[pl.ds(0, REM)], part_out.at[cid, sl])


BN = 2000
NB = N // BN


def _embed_matmul_body(x_ref, m_ref, st_ref, w_ref, root_ref, b_ref,
                       t_ref, hroot_ref):
    m = m_ref[0, 0, :].astype(_f32)
    t0 = st_ref[0, :]
    t1 = st_ref[1, :]
    h0 = x_ref[...] + t0[None, :] + m[:, None] * (t1 - t0)[None, :]
    hroot_ref[...] = jnp.dot(h0, root_ref[...],
                             preferred_element_type=_f32) + b_ref[0, :][None, :]
    for r in range(R):
        t_ref[r] = jnp.dot(h0, w_ref[r], preferred_element_type=_f32)


def _combine_matmul_body(hroot_ref, p_ref, w_ref, root_ref, b_ref,
                         t_ref, hroot2_ref):
    h = hroot_ref[...] + p_ref[0] + p_ref[1]
    h = _nan_clean(jnp.maximum(h, 0.0))
    hroot2_ref[...] = jnp.dot(h, root_ref[...],
                              preferred_element_type=_f32) + b_ref[0, :][None, :]
    for r in range(R):
        t_ref[r] = jnp.dot(h, w_ref[r], preferred_element_type=_f32)


def _pool_body(hroot_ref, p_ref, batch_ref, out_ref, sums_v, cnt_v):
    i = pl.program_id(0)
    h = hroot_ref[...] + p_ref[0] + p_ref[1]
    h = _nan_clean(jnp.maximum(h, 0.0))
    b = batch_ref[0, 0, :]
    oh = (b[:, None] == lax.broadcasted_iota(jnp.int32, (BN, G), 1)).astype(_f32)

    @pl.when(i == 0)
    def _():
        sums_v[...] = jnp.zeros((G, D), _f32)
        cnt_v[...] = jnp.zeros((G, 1), _f32)

    sums_v[...] += lax.dot_general(oh, h, (((0,), (0,)), ((), ())),
                                   preferred_element_type=_f32)
    cnt_v[...] += lax.dot_general(oh, jnp.ones((BN, 1), _f32),
                                  (((0,), (0,)), ((), ())),
                                  preferred_element_type=_f32)

    @pl.when(i == NB - 1)
    def _():
        out_ref[...] = _nan_clean(sums_v[...] / jnp.maximum(cnt_v[...], 1.0))


def _embed_matmul(x, mask3, stack_table, W, root, b2d):
    return pl.pallas_call(
        _embed_matmul_body,
        grid=(NB,),
        in_specs=[
            pl.BlockSpec((BN, D), lambda i: (i, 0)),
            pl.BlockSpec((1, 1, BN), lambda i: (i, 0, 0)),
            pl.BlockSpec((2, D), lambda i: (0, 0)),
            pl.BlockSpec((R, D, D), lambda i: (0, 0, 0)),
            pl.BlockSpec((D, D), lambda i: (0, 0)),
            pl.BlockSpec((1, D), lambda i: (0, 0)),
        ],
        out_specs=[
            pl.BlockSpec((R, BN, D), lambda i: (0, i, 0)),
            pl.BlockSpec((BN, D), lambda i: (i, 0)),
        ],
        out_shape=[
            jax.ShapeDtypeStruct((R, N, D), _f32),
            jax.ShapeDtypeStruct((N, D), _f32),
        ],
    )(x, mask3, stack_table, W, root, b2d)


def _combine_matmul(hroot, part, W, root, b2d):
    return pl.pallas_call(
        _combine_matmul_body,
        grid=(NB,),
        in_specs=[
            pl.BlockSpec((BN, D), lambda i: (i, 0)),
            pl.BlockSpec((NC, BN, D), lambda i: (0, i, 0)),
            pl.BlockSpec((R, D, D), lambda i: (0, 0, 0)),
            pl.BlockSpec((D, D), lambda i: (0, 0)),
            pl.BlockSpec((1, D), lambda i: (0, 0)),
        ],
        out_specs=[
            pl.BlockSpec((R, BN, D), lambda i: (0, i, 0)),
            pl.BlockSpec((BN, D), lambda i: (i, 0)),
        ],
        out_shape=[
            jax.ShapeDtypeStruct((R, N, D), _f32),
            jax.ShapeDtypeStruct((N, D), _f32),
        ],
    )(hroot, part, W, root, b2d)


def _pool(hroot, part, batch3):
    return pl.pallas_call(
        _pool_body,
        grid=(NB,),
        in_specs=[
            pl.BlockSpec((BN, D), lambda i: (i, 0)),
            pl.BlockSpec((NC, BN, D), lambda i: (0, i, 0)),
            pl.BlockSpec((1, 1, BN), lambda i: (i, 0, 0)),
        ],
        out_specs=pl.BlockSpec((G, D), lambda i: (0, 0)),
        out_shape=jax.ShapeDtypeStruct((G, D), _f32),
        scratch_shapes=[
            pltpu.VMEM((G, D), _f32),
            pltpu.VMEM((G, 1), _f32),
        ],
    )(hroot, part, batch3)


@jax.jit
def _run(x, edge_index, edge_type, batch, stack_mask,
         W1, root1, b1, W2, root2, b2, stack_table):
    src = edge_index[0].astype(jnp.int32)
    dst = edge_index[1].astype(jnp.int32)
    et = edge_type.astype(jnp.int32)

    gsrc3 = (et * N + src).reshape(NW, NBLK, BLK, CH)
    gdk3 = (et * N + dst).reshape(NW, NBLK, BLK, CH)
    dst3 = dst.reshape(NW, NBLK, BLK, CH)
    mask3 = stack_mask.astype(jnp.int32).reshape(NB, 1, BN)
    batch3 = batch.astype(jnp.int32).reshape(NB, 1, BN)

    zeros_k = jnp.zeros((KPT,), _f32)
    zeros_r = jnp.zeros((WRH, D), _f32)

    deg2 = _k_deg(gdk3, zeros_k)
    w3 = _k_w(gdk3, deg2[:R * N], deg2[R * N:])

    t1, hroot1 = _embed_matmul(x, mask3, stack_table, W1, root1,
                               b1.reshape(1, D))
    part1 = _k_msg(t1.reshape(R * N, D), gsrc3, dst3, w3, zeros_r)

    t2, hroot2 = _combine_matmul(hroot1, part1, W2, root2, b2.reshape(1, D))
    part2 = _k_msg(t2.reshape(R * N, D), gsrc3, dst3, w3, zeros_r)

    return _pool(hroot2, part2, batch3)


def kernel(x, edge_index, edge_type, batch, stack_mask,
           W1, root1, b1, W2, root2, b2, stack_table):
    return _run(x, edge_index, edge_type, batch, stack_mask,
                W1, root1, b1, W2, root2, b2, stack_table)


# speedup vs baseline: 29.5545x; 1.3680x over previous
import functools
import jax
import jax.numpy as jnp
from jax import lax
from jax.experimental import pallas as pl
from jax.experimental.pallas import tpu as pltpu
from jax.experimental.pallas import tpu_sc as plsc

N = 10000
E = 320000
D = 128
R = 8
G = 128

NC, NS, L = 2, 16, 16
NW = NC * NS
EPW = E // NW
CH = 80
NCHK = EPW // CH
BLK = 25
NBLK = NCHK // BLK
WRB = 624
WRH = 48
REM = N - NS * WRB
KPT = (R * N) // NS

_f32 = jnp.float32


def _nan_clean(t):
    t = jnp.where(jnp.isnan(t), 0.0, t)
    t = jnp.where(t == jnp.inf, 1000000.0, t)
    t = jnp.where(t == -jnp.inf, -1000000.0, t)
    return t


_MESH = plsc.VectorSubcoreMesh(core_axis_name="c", subcore_axis_name="s")


@functools.partial(
    pl.kernel,
    out_type=jax.ShapeDtypeStruct((NC * R * N,), _f32),
    mesh=_MESH,
    scratch_types=[
        pltpu.VMEM((NBLK, BLK, CH), jnp.int32),
        pltpu.VMEM((CH,), _f32),
        pltpu.VMEM((KPT,), _f32),
        pltpu.VMEM_SHARED((R * N,), _f32),
    ],
)
def _k_deg(gdk_hbm, zeros_hbm, deg_out, idx_v, ones_v, zb_v, acc_sh):
    cid = lax.axis_index("c")
    sid = lax.axis_index("s")
    wid = sid * NC + cid

    pltpu.sync_copy(zeros_hbm, zb_v)
    pltpu.sync_copy(zb_v, acc_sh.at[pl.ds(sid * KPT, KPT)])

    for i in range(CH // L):
        ones_v[pl.ds(i * L, L)] = jnp.ones((L,), _f32)
    pltpu.sync_copy(gdk_hbm.at[wid], idx_v)
    plsc.subcore_barrier()

    def body(b, carry):
        def inner(j, c1):
            pltpu.sync_copy(ones_v, acc_sh.at[idx_v.at[b, j]], add=True)
            return c1
        return lax.fori_loop(0, BLK, inner, carry)

    lax.fori_loop(0, NBLK, body, 0)
    plsc.subcore_barrier()

    pltpu.sync_copy(acc_sh.at[pl.ds(sid * KPT, KPT)], zb_v)
    pltpu.sync_copy(zb_v, deg_out.at[pl.ds(cid * (R * N) + sid * KPT, KPT)])


@functools.partial(
    pl.kernel,
    out_type=jax.ShapeDtypeStruct((NW, NBLK, BLK, CH), _f32),
    mesh=_MESH,
    scratch_types=[
        pltpu.VMEM((NBLK, BLK, CH), jnp.int32),
        pltpu.VMEM((CH,), _f32),
        pltpu.VMEM((CH,), _f32),
        pltpu.VMEM((NBLK, BLK, CH), _f32),
        pltpu.SemaphoreType.DMA,
    ],
)
def _k_w(gdk_hbm, deg0_hbm, deg1_hbm, w_out, idx_v, d0_v, d1_v, w_v, sem):
    cid = lax.axis_index("c")
    sid = lax.axis_index("s")
    wid = sid * NC + cid
    pltpu.sync_copy(gdk_hbm.at[wid], idx_v)

    def body(b, carry):
        def inner(j, c1):
            pltpu.async_copy(deg0_hbm.at[idx_v.at[b, j]], d0_v, sem).wait()
            pltpu.async_copy(deg1_hbm.at[idx_v.at[b, j]], d1_v, sem).wait()
            for i in range(CH // L):
                sl = pl.ds(i * L, L)
                w_v[b, j, sl] = 1.0 / (d0_v[sl] + d1_v[sl])
            return c1
        return lax.fori_loop(0, BLK, inner, carry)

    lax.fori_loop(0, NBLK, body, 0)
    pltpu.sync_copy(w_v, w_out.at[wid])


@functools.partial(
    pl.kernel,
    out_type=jax.ShapeDtypeStruct((NC, N, D), _f32),
    mesh=_MESH,
    scratch_types=[
        pltpu.VMEM((BLK, CH), jnp.int32),
        pltpu.VMEM((BLK, CH), jnp.int32),
        pltpu.VMEM((BLK, CH), _f32),
        pltpu.VMEM((2, CH, D), _f32),
        pltpu.VMEM_SHARED((N, D), _f32),
        pltpu.SemaphoreType.DMA,
        pltpu.SemaphoreType.DMA,
        pltpu.SemaphoreType.DMA,
        pltpu.SemaphoreType.DMA,
    ],
)
def _k_msg(table_hbm, gsrc_hbm, dst_hbm, w_hbm, zeros_hbm, part_out,
           gsrc_v, dst_v, w_v, rows_v, acc_sh, gs0, gs1, ss0, ss1):
    cid = lax.axis_index("c")
    sid = lax.axis_index("s")
    wid = sid * NC + cid
    gsems = (gs0, gs1)
    ssems = (ss0, ss1)
    zb = rows_v.at[0, pl.ds(0, WRH)]

    pltpu.sync_copy(zeros_hbm, zb)
    for i in range(WRB // WRH):
        pltpu.sync_copy(zb, acc_sh.at[pl.ds(sid * WRB + i * WRH, WRH)])

    @pl.when(sid == 0)
    def _():
        pltpu.sync_copy(rows_v.at[0, pl.ds(0, REM)],
                        acc_sh.at[pl.ds(NS * WRB, REM)])

    plsc.subcore_barrier()

    def issue_gather(j, buf):
        pltpu.async_copy(table_hbm.at[gsrc_v.at[j]], rows_v.at[buf],
                         gsems[buf])

    def wait_gather(buf):
        pltpu.make_async_copy(table_hbm.at[gsrc_v.at[0]], rows_v.at[buf],
                              gsems[buf]).wait()

    def issue_scatter(j, buf):
        pltpu.async_copy(rows_v.at[buf], acc_sh.at[dst_v.at[j]], ssems[buf],
                         add=True)

    def wait_scatter(buf):
        pltpu.make_async_copy(rows_v.at[buf], acc_sh.at[dst_v.at[0]],
                              ssems[buf]).wait()

    def scale(j, buf):
        def sg(g, c2):
            wv = w_v[j, pl.ds(g * L, L)]
            for el in range(L):
                e = g * L + el
                w = wv[el]
                for g2 in range(D // L):
                    sl = pl.ds(g2 * L, L)
                    rows_v[buf, e, sl] = rows_v[buf, e, sl] * w
            return c2

        lax.fori_loop(0, CH // L, sg, 0)

    def blk(b, carry):
        pltpu.sync_copy(gsrc_hbm.at[wid, b], gsrc_v)
        pltpu.sync_copy(dst_hbm.at[wid, b], dst_v)
        pltpu.sync_copy(w_hbm.at[wid, b], w_v)
        issue_gather(0, 0)

        def pair(p, c1):
            j0 = 2 * p

            @pl.when(p > 0)
            def _():
                wait_scatter(1)

            issue_gather(j0 + 1, 1)
            wait_gather(0)
            scale(j0, 0)
            issue_scatter(j0, 0)

            wait_scatter(0)
            issue_gather(j0 + 2, 0)
            wait_gather(1)
            scale(j0 + 1, 1)
            issue_scatter(j0 + 1, 1)
            return c1

        lax.fori_loop(0, BLK // 2, pair, 0)
        wait_scatter(1)
        wait_gather(0)
        scale(BLK - 1, 0)
        issue_scatter(BLK - 1, 0)
        wait_scatter(0)
        return carry

    lax.fori_loop(0, NBLK, blk, 0)
    plsc.subcore_barrier()

    for i in range(WRB // WRH):
        sl = pl.ds(sid * WRB + i * WRH, WRH)
        pltpu.sync_copy(acc_sh.at[sl], zb)
        pltpu.sync_copy(zb, part_out.at[cid, sl])

    @pl.when(sid == 0)
    def _():
        sl = pl.ds(NS * WRB, REM)
        pltpu.sync_copy(acc_sh.at[sl], rows_v.at[0, pl.ds(0, REM)])
        pltpu.sync_copy(rows_v.at[0, pl.ds(0, REM)], part_out.at[cid, sl])


BN = 2000
NB = N // BN


def _embed_matmul_body(x_ref, m_ref, st_ref, w_ref, root_ref, b_ref,
                       t_ref, hroot_ref):
    m = m_ref[0, 0, :].astype(_f32)
    t0 = st_ref[0, :]
    t1 = st_ref[1, :]
    h0 = x_ref[...] + t0[None, :] + m[:, None] * (t1 - t0)[None, :]
    hroot_ref[...] = jnp.dot(h0, root_ref[...],
                             preferred_element_type=_f32) + b_ref[0, :][None, :]
    for r in range(R):
        t_ref[r] = jnp.dot(h0, w_ref[r], preferred_element_type=_f32)


def _combine_matmul_body(hroot_ref, p_ref, w_ref, root_ref, b_ref,
                         t_ref, hroot2_ref):
    h = hroot_ref[...] + p_ref[0] + p_ref[1]
    h = _nan_clean(jnp.maximum(h, 0.0))
    hroot2_ref[...] = jnp.dot(h, root_ref[...],
                              preferred_element_type=_f32) + b_ref[0, :][None, :]
    for r in range(R):
        t_ref[r] = jnp.dot(h, w_ref[r], preferred_element_type=_f32)


def _pool_body(hroot_ref, p_ref, batch_ref, out_ref, sums_v, cnt_v):
    i = pl.program_id(0)
    h = hroot_ref[...] + p_ref[0] + p_ref[1]
    h = _nan_clean(jnp.maximum(h, 0.0))
    b = batch_ref[0, 0, :]
    oh = (b[:, None] == lax.broadcasted_iota(jnp.int32, (BN, G), 1)).astype(_f32)

    @pl.when(i == 0)
    def _():
        sums_v[...] = jnp.zeros((G, D), _f32)
        cnt_v[...] = jnp.zeros((G, 1), _f32)

    sums_v[...] += lax.dot_general(oh, h, (((0,), (0,)), ((), ())),
                                   preferred_element_type=_f32)
    cnt_v[...] += lax.dot_general(oh, jnp.ones((BN, 1), _f32),
                                  (((0,), (0,)), ((), ())),
                                  preferred_element_type=_f32)

    @pl.when(i == NB - 1)
    def _():
        out_ref[...] = _nan_clean(sums_v[...] / jnp.maximum(cnt_v[...], 1.0))


def _embed_matmul(x, mask3, stack_table, W, root, b2d):
    return pl.pallas_call(
        _embed_matmul_body,
        grid=(NB,),
        in_specs=[
            pl.BlockSpec((BN, D), lambda i: (i, 0)),
            pl.BlockSpec((1, 1, BN), lambda i: (i, 0, 0)),
            pl.BlockSpec((2, D), lambda i: (0, 0)),
            pl.BlockSpec((R, D, D), lambda i: (0, 0, 0)),
            pl.BlockSpec((D, D), lambda i: (0, 0)),
            pl.BlockSpec((1, D), lambda i: (0, 0)),
        ],
        out_specs=[
            pl.BlockSpec((R, BN, D), lambda i: (0, i, 0)),
            pl.BlockSpec((BN, D), lambda i: (i, 0)),
        ],
        out_shape=[
            jax.ShapeDtypeStruct((R, N, D), _f32),
            jax.ShapeDtypeStruct((N, D), _f32),
        ],
    )(x, mask3, stack_table, W, root, b2d)


def _combine_matmul(hroot, part, W, root, b2d):
    return pl.pallas_call(
        _combine_matmul_body,
        grid=(NB,),
        in_specs=[
            pl.BlockSpec((BN, D), lambda i: (i, 0)),
            pl.BlockSpec((NC, BN, D), lambda i: (0, i, 0)),
            pl.BlockSpec((R, D, D), lambda i: (0, 0, 0)),
            pl.BlockSpec((D, D), lambda i: (0, 0)),
            pl.BlockSpec((1, D), lambda i: (0, 0)),
        ],
        out_specs=[
            pl.BlockSpec((R, BN, D), lambda i: (0, i, 0)),
            pl.BlockSpec((BN, D), lambda i: (i, 0)),
        ],
        out_shape=[
            jax.ShapeDtypeStruct((R, N, D), _f32),
            jax.ShapeDtypeStruct((N, D), _f32),
        ],
    )(hroot, part, W, root, b2d)


def _pool(hroot, part, batch3):
    return pl.pallas_call(
        _pool_body,
        grid=(NB,),
        in_specs=[
            pl.BlockSpec((BN, D), lambda i: (i, 0)),
            pl.BlockSpec((NC, BN, D), lambda i: (0, i, 0)),
            pl.BlockSpec((1, 1, BN), lambda i: (i, 0, 0)),
        ],
        out_specs=pl.BlockSpec((G, D), lambda i: (0, 0)),
        out_shape=jax.ShapeDtypeStruct((G, D), _f32),
        scratch_shapes=[
            pltpu.VMEM((G, D), _f32),
            pltpu.VMEM((G, 1), _f32),
        ],
    )(hroot, part, batch3)


@jax.jit
def _run(x, edge_index, edge_type, batch, stack_mask,
         W1, root1, b1, W2, root2, b2, stack_table):
    src = edge_index[0].astype(jnp.int32)
    dst = edge_index[1].astype(jnp.int32)
    et = edge_type.astype(jnp.int32)

    gsrc3 = (et * N + src).reshape(NW, NBLK, BLK, CH)
    gdk3 = (et * N + dst).reshape(NW, NBLK, BLK, CH)
    dst3 = dst.reshape(NW, NBLK, BLK, CH)
    mask3 = stack_mask.astype(jnp.int32).reshape(NB, 1, BN)
    batch3 = batch.astype(jnp.int32).reshape(NB, 1, BN)

    zeros_k = jnp.zeros((KPT,), _f32)
    zeros_r = jnp.zeros((WRH, D), _f32)

    deg2 = _k_deg(gdk3, zeros_k)
    w3 = _k_w(gdk3, deg2[:R * N], deg2[R * N:])

    t1, hroot1 = _embed_matmul(x, mask3, stack_table, W1, root1,
                               b1.reshape(1, D))
    part1 = _k_msg(t1.reshape(R * N, D), gsrc3, dst3, w3, zeros_r)

    t2, hroot2 = _combine_matmul(hroot1, part1, W2, root2, b2.reshape(1, D))
    part2 = _k_msg(t2.reshape(R * N, D), gsrc3, dst3, w3, zeros_r)

    return _pool(hroot2, part2, batch3)


def kernel(x, edge_index, edge_type, batch, stack_mask,
           W1, root1, b1, W2, root2, b2, stack_table):
    return _run(x, edge_index, edge_type, batch, stack_mask,
                W1, root1, b1, W2, root2, b2, stack_table)


# speedup vs baseline: 41.3487x; 1.3991x over previous
import functools
import jax
import jax.numpy as jnp
from jax import lax
from jax.experimental import pallas as pl
from jax.experimental.pallas import tpu as pltpu
from jax.experimental.pallas import tpu_sc as plsc

N = 10000
E = 320000
D = 128
R = 8
G = 128

NC, NS, L = 2, 16, 16
NW = NC * NS
EPW = E // NW
CH = 80
NCHK = EPW // CH
BLK = 25
NBLK = NCHK // BLK
NB2 = (E // NS) // (BLK * CH)
WRB = 624
WRH = 48
REM = N - NS * WRB
KPT = (R * N) // NS

_f32 = jnp.float32


def _nan_clean(t):
    t = jnp.where(jnp.isnan(t), 0.0, t)
    t = jnp.where(t == jnp.inf, 1000000.0, t)
    t = jnp.where(t == -jnp.inf, -1000000.0, t)
    return t


_MESH = plsc.VectorSubcoreMesh(core_axis_name="c", subcore_axis_name="s")


@functools.partial(
    pl.kernel,
    out_type=jax.ShapeDtypeStruct((NW, NBLK, BLK, CH), _f32),
    mesh=_MESH,
    scratch_types=[
        pltpu.VMEM((NB2, BLK, CH), jnp.int32),
        pltpu.VMEM((NBLK, BLK, CH), jnp.int32),
        pltpu.VMEM((CH,), _f32),
        pltpu.VMEM((CH,), _f32),
        pltpu.VMEM((NBLK, BLK, CH), _f32),
        pltpu.VMEM((KPT,), _f32),
        pltpu.VMEM_SHARED((R * N,), _f32),
        pltpu.SemaphoreType.DMA,
        pltpu.SemaphoreType.DMA,
    ],
)
def _k_degw(gdkc_hbm, gdkw_hbm, zeros_hbm, w_out,
            idxc_v, idxw_v, ones_v, d_v, w_v, zb_v, acc_sh, ssem, gsem):
    cid = lax.axis_index("c")
    sid = lax.axis_index("s")
    wid = sid * NC + cid

    pltpu.sync_copy(zeros_hbm, zb_v)
    pltpu.sync_copy(zb_v, acc_sh.at[pl.ds(sid * KPT, KPT)])

    for i in range(CH // L):
        ones_v[pl.ds(i * L, L)] = jnp.ones((L,), _f32)
    pltpu.sync_copy(gdkc_hbm.at[sid], idxc_v)
    pltpu.sync_copy(gdkw_hbm.at[wid], idxw_v)
    plsc.subcore_barrier()

    def cnt_blk(b, carry):
        def fire(j, c1):
            pltpu.async_copy(ones_v, acc_sh.at[idxc_v.at[b, j]], ssem,
                             add=True)
            return c1

        def drain(j, c1):
            pltpu.make_async_copy(ones_v, acc_sh.at[idxc_v.at[0, 0]],
                                  ssem).wait()
            return c1

        lax.fori_loop(0, BLK, fire, carry)
        return lax.fori_loop(0, BLK, drain, carry)

    lax.fori_loop(0, NB2, cnt_blk, 0)
    plsc.subcore_barrier()

    def w_blk(b, carry):
        def inner(j, c1):
            pltpu.async_copy(acc_sh.at[idxw_v.at[b, j]], d_v, gsem).wait()
            for i in range(CH // L):
                sl = pl.ds(i * L, L)
                w_v[b, j, sl] = 1.0 / d_v[sl]
            return c1
        return lax.fori_loop(0, BLK, inner, carry)

    lax.fori_loop(0, NBLK, w_blk, 0)
    pltpu.sync_copy(w_v, w_out.at[wid])


@functools.partial(
    pl.kernel,
    out_type=jax.ShapeDtypeStruct((NC, N, D), _f32),
    mesh=_MESH,
    scratch_types=[
        pltpu.VMEM((BLK, CH), jnp.int32),
        pltpu.VMEM((BLK, CH), jnp.int32),
        pltpu.VMEM((BLK, CH), _f32),
        pltpu.VMEM((2, CH, D), _f32),
        pltpu.VMEM_SHARED((N, D), _f32),
        pltpu.SemaphoreType.DMA,
        pltpu.SemaphoreType.DMA,
        pltpu.SemaphoreType.DMA,
        pltpu.SemaphoreType.DMA,
    ],
)
def _k_msg(table_hbm, gsrc_hbm, dst_hbm, w_hbm, zeros_hbm, part_out,
           gsrc_v, dst_v, w_v, rows_v, acc_sh, gs0, gs1, ss0, ss1):
    cid = lax.axis_index("c")
    sid = lax.axis_index("s")
    wid = sid * NC + cid
    gsems = (gs0, gs1)
    ssems = (ss0, ss1)
    zb = rows_v.at[0, pl.ds(0, WRH)]

    pltpu.sync_copy(zeros_hbm, zb)
    for i in range(WRB // WRH):
        pltpu.sync_copy(zb, acc_sh.at[pl.ds(sid * WRB + i * WRH, WRH)])

    @pl.when(sid == 0)
    def _():
        pltpu.sync_copy(rows_v.at[0, pl.ds(0, REM)],
                        acc_sh.at[pl.ds(NS * WRB, REM)])

    plsc.subcore_barrier()

    def issue_gather(j, buf):
        pltpu.async_copy(table_hbm.at[gsrc_v.at[j]], rows_v.at[buf],
                         gsems[buf])

    def wait_gather(buf):
        pltpu.make_async_copy(table_hbm.at[gsrc_v.at[0]], rows_v.at[buf],
                              gsems[buf]).wait()

    def issue_scatter(j, buf):
        pltpu.async_copy(rows_v.at[buf], acc_sh.at[dst_v.at[j]], ssems[buf],
                         add=True)

    def wait_scatter(buf):
        pltpu.make_async_copy(rows_v.at[buf], acc_sh.at[dst_v.at[0]],
                              ssems[buf]).wait()

    def scale(j, buf):
        def sg(g, c2):
            wv = w_v[j, pl.ds(g * L, L)]
            for el in range(L):
                e = g * L + el
                w = wv[el]
                for g2 in range(D // L):
                    sl = pl.ds(g2 * L, L)
                    rows_v[buf, e, sl] = rows_v[buf, e, sl] * w
            return c2

        lax.fori_loop(0, CH // L, sg, 0)

    def blk(b, carry):
        pltpu.sync_copy(gsrc_hbm.at[wid, b], gsrc_v)
        pltpu.sync_copy(dst_hbm.at[wid, b], dst_v)
        pltpu.sync_copy(w_hbm.at[wid, b], w_v)
        issue_gather(0, 0)

        def pair(p, c1):
            j0 = 2 * p

            @pl.when(p > 0)
            def _():
                wait_scatter(1)

            issue_gather(j0 + 1, 1)
            wait_gather(0)
            scale(j0, 0)
            issue_scatter(j0, 0)

            wait_scatter(0)
            issue_gather(j0 + 2, 0)
            wait_gather(1)
            scale(j0 + 1, 1)
            issue_scatter(j0 + 1, 1)
            return c1

        lax.fori_loop(0, BLK // 2, pair, 0)
        wait_scatter(1)
        wait_gather(0)
        scale(BLK - 1, 0)
        issue_scatter(BLK - 1, 0)
        wait_scatter(0)
        return carry

    lax.fori_loop(0, NBLK, blk, 0)
    plsc.subcore_barrier()

    for i in range(WRB // WRH):
        sl = pl.ds(sid * WRB + i * WRH, WRH)
        pltpu.sync_copy(acc_sh.at[sl], zb)
        pltpu.sync_copy(zb, part_out.at[cid, sl])

    @pl.when(sid == 0)
    def _():
        sl = pl.ds(NS * WRB, REM)
        pltpu.sync_copy(acc_sh.at[sl], rows_v.at[0, pl.ds(0, REM)])
        pltpu.sync_copy(rows_v.at[0, pl.ds(0, REM)], part_out.at[cid, sl])


BN = 2000
NB = N // BN


def _embed_matmul_body(x_ref, m_ref, st_ref, w_ref, root_ref, b_ref,
                       t_ref, hroot_ref):
    m = m_ref[0, 0, :].astype(_f32)
    t0 = st_ref[0, :]
    t1 = st_ref[1, :]
    h0 = x_ref[...] + t0[None, :] + m[:, None] * (t1 - t0)[None, :]
    hroot_ref[...] = jnp.dot(h0, root_ref[...],
                             preferred_element_type=_f32) + b_ref[0, :][None, :]
    for r in range(R):
        t_ref[r] = jnp.dot(h0, w_ref[r], preferred_element_type=_f32)


def _combine_matmul_body(hroot_ref, p_ref, w_ref, root_ref, b_ref,
                         t_ref, hroot2_ref):
    h = hroot_ref[...] + p_ref[0] + p_ref[1]
    h = _nan_clean(jnp.maximum(h, 0.0))
    hroot2_ref[...] = jnp.dot(h, root_ref[...],
                              preferred_element_type=_f32) + b_ref[0, :][None, :]
    for r in range(R):
        t_ref[r] = jnp.dot(h, w_ref[r], preferred_element_type=_f32)


def _pool_body(hroot_ref, p_ref, batch_ref, out_ref, sums_v, cnt_v):
    i = pl.program_id(0)
    h = hroot_ref[...] + p_ref[0] + p_ref[1]
    h = _nan_clean(jnp.maximum(h, 0.0))
    b = batch_ref[0, 0, :]
    oh = (b[:, None] == lax.broadcasted_iota(jnp.int32, (BN, G), 1)).astype(_f32)

    @pl.when(i == 0)
    def _():
        sums_v[...] = jnp.zeros((G, D), _f32)
        cnt_v[...] = jnp.zeros((G, 1), _f32)

    sums_v[...] += lax.dot_general(oh, h, (((0,), (0,)), ((), ())),
                                   preferred_element_type=_f32)
    cnt_v[...] += lax.dot_general(oh, jnp.ones((BN, 1), _f32),
                                  (((0,), (0,)), ((), ())),
                                  preferred_element_type=_f32)

    @pl.when(i == NB - 1)
    def _():
        out_ref[...] = _nan_clean(sums_v[...] / jnp.maximum(cnt_v[...], 1.0))


def _embed_matmul(x, mask3, stack_table, W, root, b2d):
    return pl.pallas_call(
        _embed_matmul_body,
        grid=(NB,),
        in_specs=[
            pl.BlockSpec((BN, D), lambda i: (i, 0)),
            pl.BlockSpec((1, 1, BN), lambda i: (i, 0, 0)),
            pl.BlockSpec((2, D), lambda i: (0, 0)),
            pl.BlockSpec((R, D, D), lambda i: (0, 0, 0)),
            pl.BlockSpec((D, D), lambda i: (0, 0)),
            pl.BlockSpec((1, D), lambda i: (0, 0)),
        ],
        out_specs=[
            pl.BlockSpec((R, BN, D), lambda i: (0, i, 0)),
            pl.BlockSpec((BN, D), lambda i: (i, 0)),
        ],
        out_shape=[
            jax.ShapeDtypeStruct((R, N, D), _f32),
            jax.ShapeDtypeStruct((N, D), _f32),
        ],
    )(x, mask3, stack_table, W, root, b2d)


def _combine_matmul(hroot, part, W, root, b2d):
    return pl.pallas_call(
        _combine_matmul_body,
        grid=(NB,),
        in_specs=[
            pl.BlockSpec((BN, D), lambda i: (i, 0)),
            pl.BlockSpec((NC, BN, D), lambda i: (0, i, 0)),
            pl.BlockSpec((R, D, D), lambda i: (0, 0, 0)),
            pl.BlockSpec((D, D), lambda i: (0, 0)),
            pl.BlockSpec((1, D), lambda i: (0, 0)),
        ],
        out_specs=[
            pl.BlockSpec((R, BN, D), lambda i: (0, i, 0)),
            pl.BlockSpec((BN, D), lambda i: (i, 0)),
        ],
        out_shape=[
            jax.ShapeDtypeStruct((R, N, D), _f32),
            jax.ShapeDtypeStruct((N, D), _f32),
        ],
    )(hroot, part, W, root, b2d)


def _pool(hroot, part, batch3):
    return pl.pallas_call(
        _pool_body,
        grid=(NB,),
        in_specs=[
            pl.BlockSpec((BN, D), lambda i: (i, 0)),
            pl.BlockSpec((NC, BN, D), lambda i: (0, i, 0)),
            pl.BlockSpec((1, 1, BN), lambda i: (i, 0, 0)),
        ],
        out_specs=pl.BlockSpec((G, D), lambda i: (0, 0)),
        out_shape=jax.ShapeDtypeStruct((G, D), _f32),
        scratch_shapes=[
            pltpu.VMEM((G, D), _f32),
            pltpu.VMEM((G, 1), _f32),
        ],
    )(hroot, part, batch3)


@jax.jit
def _run(x, edge_index, edge_type, batch, stack_mask,
         W1, root1, b1, W2, root2, b2, stack_table):
    src = edge_index[0].astype(jnp.int32)
    dst = edge_index[1].astype(jnp.int32)
    et = edge_type.astype(jnp.int32)

    gsrc3 = (et * N + src).reshape(NW, NBLK, BLK, CH)
    gdk = et * N + dst
    dst3 = dst.reshape(NW, NBLK, BLK, CH)
    mask3 = stack_mask.astype(jnp.int32).reshape(NB, 1, BN)
    batch3 = batch.astype(jnp.int32).reshape(NB, 1, BN)

    zeros_k = jnp.zeros((KPT,), _f32)
    zeros_r = jnp.zeros((WRH, D), _f32)

    w3 = _k_degw(gdk.reshape(NS, NB2, BLK, CH), gdk.reshape(NW, NBLK, BLK, CH),
                 zeros_k)

    t1, hroot1 = _embed_matmul(x, mask3, stack_table, W1, root1,
                               b1.reshape(1, D))
    part1 = _k_msg(t1.reshape(R * N, D), gsrc3, dst3, w3, zeros_r)

    t2, hroot2 = _combine_matmul(hroot1, part1, W2, root2, b2.reshape(1, D))
    part2 = _k_msg(t2.reshape(R * N, D), gsrc3, dst3, w3, zeros_r)

    return _pool(hroot2, part2, batch3)


def kernel(x, edge_index, edge_type, batch, stack_mask,
           W1, root1, b1, W2, root2, b2, stack_table):
    return _run(x, edge_index, edge_type, batch, stack_mask,
                W1, root1, b1, W2, root2, b2, stack_table)
